# speedup vs baseline: 1.2663x; 1.2663x over previous
import functools
import jax
import jax.numpy as jnp
from jax.experimental import pallas as pl
from jax.experimental.pallas import tpu as pltpu

NEG = jnp.finfo(jnp.float32).min


def _body(adj21, adj12, f2, f1, x1, x2, w1s, w1n, w2s, w2n,
          out1, out2, m_acc, h_acc, s_acc, *, n_col_blocks):
    c = pl.program_id(1)

    a21 = adj21[...]
    a12 = adj12[...]
    mask1 = a21 != 0
    vals = jnp.where(mask1, f2[...], NEG)
    m = jnp.max(vals, axis=1, keepdims=True)
    h = jnp.max(a21, axis=1, keepdims=True)
    s = jnp.sum(jnp.where(a12 != 0, f1[...], 0.0), axis=1, keepdims=True)

    @pl.when(c == 0)
    def _init():
        m_acc[...] = m
        h_acc[...] = h
        s_acc[...] = s

    @pl.when(c > 0)
    def _accum():
        m_acc[...] = jnp.maximum(m_acc[...], m)
        h_acc[...] = jnp.maximum(h_acc[...], h)
        s_acc[...] = s_acc[...] + s

    @pl.when(c == n_col_blocks - 1)
    def _finalize():
        scal1 = jnp.where(h_acc[...] > 0, m_acc[...], 0.0)
        scal2 = s_acc[...]
        wsum1 = jnp.sum(w1n[...], axis=1)
        wsum2 = jnp.sum(w2n[...], axis=1)
        o1 = jnp.dot(x1[...], w1s[...].T, preferred_element_type=jnp.float32)
        o2 = jnp.dot(x2[...], w2s[...].T, preferred_element_type=jnp.float32)
        out1[...] = jnp.maximum(o1 + scal1 * wsum1[None, :], 0.0)
        out2[...] = jnp.maximum(o2 + scal2 * wsum2[None, :], 0.0)


def kernel(node_feats1, node_feats2, adj_1to2, adj_2to1,
           W1_self, W1_neigh, W2_self, W2_neigh):
    n1, d_in = node_feats1.shape
    n2, _ = node_feats2.shape
    d_out = W1_self.shape[0]

    br = min(256, n1)
    bc = min(2048, n2)
    nr = n1 // br
    nc = n2 // bc

    f2_row = node_feats2[:, 0].reshape(1, n2)
    f1_row = node_feats1[:, 0].reshape(1, n1)

    grid = (nr, nc)
    out1, out2 = pl.pallas_call(
        functools.partial(_body, n_col_blocks=nc),
        grid=grid,
        in_specs=[
            pl.BlockSpec((br, bc), lambda r, c: (r, c)),
            pl.BlockSpec((br, bc), lambda r, c: (r, c)),
            pl.BlockSpec((1, bc), lambda r, c: (0, c)),
            pl.BlockSpec((1, bc), lambda r, c: (0, c)),
            pl.BlockSpec((br, d_in), lambda r, c: (r, 0)),
            pl.BlockSpec((br, d_in), lambda r, c: (r, 0)),
            pl.BlockSpec((d_out, d_in), lambda r, c: (0, 0)),
            pl.BlockSpec((d_out, d_in), lambda r, c: (0, 0)),
            pl.BlockSpec((d_out, d_in), lambda r, c: (0, 0)),
            pl.BlockSpec((d_out, d_in), lambda r, c: (0, 0)),
        ],
        out_specs=[
            pl.BlockSpec((br, d_out), lambda r, c: (r, 0)),
            pl.BlockSpec((br, d_out), lambda r, c: (r, 0)),
        ],
        out_shape=[
            jax.ShapeDtypeStruct((n1, d_out), jnp.float32),
            jax.ShapeDtypeStruct((n2, d_out), jnp.float32),
        ],
        scratch_shapes=[
            pltpu.VMEM((br, 1), jnp.float32),
            pltpu.VMEM((br, 1), jnp.int32),
            pltpu.VMEM((br, 1), jnp.float32),
        ],
        compiler_params=pltpu.CompilerParams(
            dimension_semantics=("parallel", "arbitrary"),
        ),
    )(adj_2to1, adj_1to2, f2_row, f1_row, node_feats1, node_feats2,
      W1_self, W1_neigh, W2_self, W2_neigh)
    return out1, out2
